# Initial kernel scaffold; baseline (speedup 1.0000x reference)
import jax
import jax.numpy as jnp
from jax.experimental import pallas as pl


def kernel(query_embeddings, context_embeddings, W, b):
    raise NotImplementedError("write your pallas kernel here")


# speedup vs baseline: 14.8048x; 14.8048x over previous
import functools

import jax
import jax.numpy as jnp
from jax.experimental import pallas as pl

B, S, C, D, TOPN = 16, 32, 4096, 256, 8


def _topk_kernel(q_ref, ctx_ref, w_ref, b_ref, dist_out_ref, idx_out_ref):
    q = q_ref[0]
    w = w_ref[...]
    bias = b_ref[...]
    qp = jax.lax.dot_general(q, w, (((1,), (1,)), ((), ()))) + bias

    ctx = ctx_ref[0]
    a2 = jnp.sum(qp * qp, axis=-1, keepdims=True)
    b2 = jnp.sum(ctx * ctx, axis=-1)
    ab = jax.lax.dot_general(qp, ctx, (((1,), (1,)), ((), ())))
    d2 = jnp.maximum(a2 + b2[None, :] - 2.0 * ab, 0.0)
    dist = jnp.sqrt(d2)

    iota = jax.lax.broadcasted_iota(jnp.int32, (S, C), 1)
    vals = dist
    top_vals = []
    top_idx = []
    for _ in range(TOPN):
        mv = jnp.min(vals, axis=1, keepdims=True)
        eq = vals == mv
        mi = jnp.min(jnp.where(eq, iota, C), axis=1, keepdims=True)
        top_vals.append(mv)
        top_idx.append(mi)
        vals = jnp.where(iota == mi, jnp.float32(jnp.inf), vals)
    dist_out_ref[0] = jnp.concatenate(top_vals, axis=1)
    idx_out_ref[0] = jnp.concatenate(top_idx, axis=1)


@jax.jit
def kernel(query_embeddings, context_embeddings, W, b):
    bias2d = b.reshape(1, D)
    grid = (B,)
    out_dist, out_idx = pl.pallas_call(
        _topk_kernel,
        grid=grid,
        in_specs=[
            pl.BlockSpec((1, S, D), lambda i: (i, 0, 0)),
            pl.BlockSpec((1, C, D), lambda i: (i, 0, 0)),
            pl.BlockSpec((D, D), lambda i: (0, 0)),
            pl.BlockSpec((1, D), lambda i: (0, 0)),
        ],
        out_specs=[
            pl.BlockSpec((1, S, TOPN), lambda i: (i, 0, 0)),
            pl.BlockSpec((1, S, TOPN), lambda i: (i, 0, 0)),
        ],
        out_shape=[
            jax.ShapeDtypeStruct((B, S, TOPN), jnp.float32),
            jax.ShapeDtypeStruct((B, S, TOPN), jnp.int32),
        ],
    )(query_embeddings, context_embeddings, W, bias2d)
    return (out_dist, out_idx)


# speedup vs baseline: 14.8139x; 1.0006x over previous
import functools

import jax
import jax.numpy as jnp
from jax.experimental import pallas as pl
from jax.experimental.pallas import tpu as pltpu

B, S, C, D, TOPN = 16, 32, 4096, 256, 8


def _topk_kernel(q_ref, ctx_ref, w_ref, b_ref, dist_out_ref, idx_out_ref):
    q = q_ref[0]
    w = w_ref[...]
    bias = b_ref[...]
    qp = jax.lax.dot_general(q, w, (((1,), (1,)), ((), ()))) + bias

    ctx = ctx_ref[0]
    a2 = jnp.sum(qp * qp, axis=-1, keepdims=True)
    b2 = jnp.sum(ctx * ctx, axis=-1)
    ab = jax.lax.dot_general(qp, ctx, (((1,), (1,)), ((), ())))
    d2 = jnp.maximum(a2 + b2[None, :] - 2.0 * ab, 0.0)
    dist = jnp.sqrt(d2)

    iota = jax.lax.broadcasted_iota(jnp.int32, (S, C), 1)
    vals = dist
    top_vals = []
    top_idx = []
    for _ in range(TOPN):
        mv = jnp.min(vals, axis=1, keepdims=True)
        eq = vals == mv
        mi = jnp.min(jnp.where(eq, iota, C), axis=1, keepdims=True)
        top_vals.append(mv)
        top_idx.append(mi)
        vals = jnp.where(iota == mi, jnp.float32(jnp.inf), vals)
    dist_out_ref[0] = jnp.concatenate(top_vals, axis=1)
    idx_out_ref[0] = jnp.concatenate(top_idx, axis=1)


@jax.jit
def kernel(query_embeddings, context_embeddings, W, b):
    bias2d = b.reshape(1, D)
    grid = (B,)
    out_dist, out_idx = pl.pallas_call(
        _topk_kernel,
        grid=grid,
        in_specs=[
            pl.BlockSpec((1, S, D), lambda i: (i, 0, 0)),
            pl.BlockSpec((1, C, D), lambda i: (i, 0, 0)),
            pl.BlockSpec((D, D), lambda i: (0, 0)),
            pl.BlockSpec((1, D), lambda i: (0, 0)),
        ],
        out_specs=[
            pl.BlockSpec((1, S, TOPN), lambda i: (i, 0, 0)),
            pl.BlockSpec((1, S, TOPN), lambda i: (i, 0, 0)),
        ],
        out_shape=[
            jax.ShapeDtypeStruct((B, S, TOPN), jnp.float32),
            jax.ShapeDtypeStruct((B, S, TOPN), jnp.int32),
        ],
        compiler_params=pltpu.CompilerParams(
            dimension_semantics=("parallel",),
        ),
    )(query_embeddings, context_embeddings, W, bias2d)
    return (out_dist, out_idx)
